# Initial kernel scaffold; baseline (speedup 1.0000x reference)
import jax
import jax.numpy as jnp
from jax.experimental import pallas as pl


def kernel(database, query, nn_index, nn_count, nn_dist):
    raise NotImplementedError("write your pallas kernel here")


# speedup vs baseline: 13.2224x; 13.2224x over previous
import functools
import math

import jax
import jax.numpy as jnp
import numpy as np
from jax import lax
from jax.experimental import pallas as pl
from jax.experimental.pallas import tpu as pltpu
from jax.experimental.pallas import tpu_sc as plsc

_ATAN_C = (0.9999999975460202, -0.3333328229551182, 0.19998230640374604,
           -0.14261573680264983, 0.10940198965043517, -0.08372063947828604,
           0.057463557846092916, -0.030717508903231272, 0.010680719446080988,
           -0.0017437011439092583)
_ACOS_C = (1.5707963231771758, -0.21460122624982533, 0.08903147164326626,
           -0.050604003732084984, 0.032611384067433316, -0.02081215069447393,
           0.011126960102177536, -0.004034589383599449, 0.0006993961325743523)

_PI = np.float32(math.pi)
_HALF_PI = np.float32(math.pi / 2)
_C_AZ = np.float32(4 / (2 * math.pi))
_C_EL = np.float32(2 / math.pi)
_RADIUS = np.float32(0.05)
_R_CLIP = np.float32(2 - 1e-6)

_B, _N, _M, _K = 8, 32768, 8192, 32
_NW = 32
_WPB = _NW // _B
_MQ = _M // _WPB
_CM = 16
_CE = _CM * _K
_NCHUNK = _MQ // _CM
_VPC = _CE // 16


def _horner(t, coeffs):
    acc = jnp.full((16,), coeffs[-1], dtype=jnp.float32)
    for c in reversed(coeffs[:-1]):
        acc = acc * t + np.float32(c)
    return acc


def _sqrt16(v):
    i = plsc.bitcast(v, jnp.int32)
    i = np.int32(0x5F3759DF) - (i >> 1)
    y = plsc.bitcast(i, jnp.float32)
    half_v = np.float32(0.5) * v
    for _ in range(3):
        y = y * (np.float32(1.5) - (half_v * y) * y)
    return v * y


def _compute_vreg(nx, ny, nz, qx, qy, qz, d):
    x = nx - qx
    y = ny - qy
    z = nz - qz

    ax = jnp.abs(x)
    ay = jnp.abs(y)
    num = jnp.minimum(ax, ay)
    den = jnp.maximum(ax, ay)
    q0 = num / den
    q0 = jnp.where(den > 0, q0, np.float32(0.0))
    t = q0 * q0
    r0 = q0 * _horner(t, _ATAN_C)
    r1 = jnp.where(ay > ax, _HALF_PI - r0, r0)
    r2 = jnp.where(x < 0, _PI - r1, r1)
    theta = jnp.where(y < 0, -r2, r2)
    azb = (theta + _PI) * _C_AZ

    d1 = d + np.float32(1e-8)
    u = z / d1
    u = jnp.clip(u, np.float32(-1.0), np.float32(1.0))
    s = jnp.abs(u)
    sq = _sqrt16(np.float32(1.0) - s)
    acp = sq * _horner(s, _ACOS_C)
    ac = jnp.where(u < 0, _PI - acp, acp)
    elb = ac * _C_EL

    rb = jnp.clip(d / _RADIUS, np.float32(0.0), _R_CLIP)

    af = azb.astype(jnp.int32)
    ef = elb.astype(jnp.int32)
    rf = rb.astype(jnp.int32)
    a_frac = azb - af.astype(jnp.float32)
    e_frac = elb - ef.astype(jnp.float32)
    r_frac = rb - rf.astype(jnp.float32)

    ca = np.float32(1.0) - a_frac
    ce = np.float32(1.0) - e_frac
    cr = np.float32(1.0) - r_frac
    w00 = ca * ce
    w10 = a_frac * ce
    w01 = ca * e_frac
    w11 = a_frac * e_frac
    ws = (w00 * cr, w10 * cr, w01 * cr, w11 * cr,
          w00 * r_frac, w10 * r_frac, w01 * r_frac, w11 * r_frac)

    three = np.int32(3)
    a0 = (af & three) << 2
    a1 = ((af + 1) & three) << 2
    e0 = jnp.minimum(ef, 1) << 1
    rr = jnp.minimum(rf, 1)
    p00 = a0 + e0
    p10 = a1 + e0
    p01 = a0 + 2
    p11 = a1 + 2
    idxs = (p00 + rr, p10 + rr, p01 + rr, p11 + rr,
            p00 + 1, p10 + 1, p01 + 1, p11 + 1)
    return idxs, ws


def _make_kernel():
    mesh = plsc.VectorSubcoreMesh(core_axis_name="c", subcore_axis_name="s")

    @functools.partial(
        pl.kernel,
        out_type=[jax.ShapeDtypeStruct((_NW, _NCHUNK, _CE * 8), jnp.int32),
                  jax.ShapeDtypeStruct((_NW, _NCHUNK, _CE * 8), jnp.float32)],
        mesh=mesh,
        compiler_params=pltpu.CompilerParams(needs_layout_passes=False),
        scratch_types=[
            pltpu.VMEM((_N * 3,), jnp.float32),
            pltpu.VMEM((_MQ * 3,), jnp.float32),
            pltpu.VMEM((_CE,), jnp.int32),
            pltpu.VMEM((_CE,), jnp.float32),
            pltpu.VMEM((_CE * 8,), jnp.int32),
            pltpu.VMEM((_CE * 8,), jnp.float32),
        ],
    )
    def fuzzy(db_hbm, q_hbm, idx_hbm, dist_hbm, oi_hbm, ow_hbm,
              db_v, q_v, idx_v, dist_v, oi_v, ow_v):
        w = lax.axis_index("s") * 2 + lax.axis_index("c")
        b = w >> 2
        pltpu.sync_copy(db_hbm.at[b], db_v)
        pltpu.sync_copy(q_hbm.at[w], q_v)
        lane = lax.iota(jnp.int32, 16)

        def chunk_body(c, carry):
            pltpu.sync_copy(idx_hbm.at[w, c], idx_v)
            pltpu.sync_copy(dist_hbm.at[w, c], dist_v)

            def vreg_body(v, carry2):
                base = v * 16
                el = base + lane
                nidx = idx_v[pl.ds(base, 16)]
                i3 = nidx * 3
                nx = plsc.load_gather(db_v, [i3])
                ny = plsc.load_gather(db_v, [i3 + 1])
                nz = plsc.load_gather(db_v, [i3 + 2])
                m3 = ((c * _CM) + (el >> 5)) * 3
                qx = plsc.load_gather(q_v, [m3])
                qy = plsc.load_gather(q_v, [m3 + 1])
                qz = plsc.load_gather(q_v, [m3 + 2])
                d = dist_v[pl.ds(base, 16)]
                idxs, ws = _compute_vreg(nx, ny, nz, qx, qy, qz, d)
                opos = el << 3
                for j in range(8):
                    plsc.store_scatter(oi_v, [opos + j], idxs[j])
                    plsc.store_scatter(ow_v, [opos + j], ws[j])
                return carry2

            lax.fori_loop(0, _VPC, vreg_body, 0)
            pltpu.sync_copy(oi_v, oi_hbm.at[w, c])
            pltpu.sync_copy(ow_v, ow_hbm.at[w, c])
            return carry

        lax.fori_loop(0, _NCHUNK, chunk_body, 0)

    return fuzzy


_FUZZY = _make_kernel()


def kernel(database, query, nn_index, nn_count, nn_dist):
    del nn_count
    B, M, K = nn_index.shape
    db2 = database.reshape(_B, _N * 3)
    q2 = query.reshape(_NW, _MQ * 3)
    idx2 = nn_index.reshape(_NW, _NCHUNK, _CE)
    dist2 = nn_dist.reshape(_NW, _NCHUNK, _CE)
    oi, ow = _FUZZY(db2, q2, idx2, dist2)
    return oi.reshape(B, M, K, 8), ow.reshape(B, M, K, 8)
